# Initial kernel scaffold; baseline (speedup 1.0000x reference)
import jax
import jax.numpy as jnp
from jax.experimental import pallas as pl


def kernel(x, W1, b1, W2, b2):
    raise NotImplementedError("write your pallas kernel here")


# speedup vs baseline: 1.8178x; 1.8178x over previous
import functools

import jax
import jax.numpy as jnp
from jax.experimental import pallas as pl
from jax.experimental.pallas import tpu as pltpu

TOKENS = 8192
D_MODEL = 4096
D_HID = 4096
N_EXPERTS = 64

TOK_BLK = 1024
HID_BLK = 512


def _gelu(v):
    return 0.5 * v * (1.0 + jax.lax.erf(v / jnp.sqrt(2.0).astype(v.dtype)))


def _router_kernel(x_ref, w1_ref, b1_ref, w2_ref, b2_ref,
                   wout_ref, iout_ref, acc_ref):
    j = pl.program_id(1)
    nj = pl.num_programs(1)

    xb = x_ref[...].astype(jnp.bfloat16)
    w1b = w1_ref[...].astype(jnp.bfloat16)
    h = jax.lax.dot_general(
        xb, w1b, (((1,), (1,)), ((), ())),
        preferred_element_type=jnp.float32)
    h = _gelu(h + b1_ref[...])
    part = jax.lax.dot_general(
        h.astype(jnp.bfloat16), w2_ref[...].astype(jnp.bfloat16),
        (((1,), (1,)), ((), ())),
        preferred_element_type=jnp.float32)

    @pl.when(j == 0)
    def _init():
        acc_ref[...] = part

    @pl.when(j > 0)
    def _accum():
        acc_ref[...] += part

    @pl.when(j == nj - 1)
    def _epilogue():
        logits = _gelu(acc_ref[...] + b2_ref[...])
        idx = jax.lax.broadcasted_iota(jnp.int32, logits.shape, 1)
        m1 = jnp.max(logits, axis=1, keepdims=True)
        i1 = jnp.min(jnp.where(logits == m1, idx, N_EXPERTS),
                     axis=1, keepdims=True)
        masked = jnp.where(idx == i1, -jnp.inf, logits)
        m2 = jnp.max(masked, axis=1, keepdims=True)
        i2 = jnp.min(jnp.where(masked == m2, idx, N_EXPERTS),
                     axis=1, keepdims=True)
        e2 = jnp.exp(m2 - m1)
        denom = 1.0 + e2
        wout_ref[...] = jnp.concatenate([1.0 / denom, e2 / denom], axis=1)
        iout_ref[...] = jnp.concatenate([i1, i2], axis=1)


@jax.jit
def kernel(x, W1, b1, W2, b2):
    n_tok = TOKENS // TOK_BLK
    n_hid = D_HID // HID_BLK
    b1r = b1.reshape(1, D_HID)
    b2r = b2.reshape(1, N_EXPERTS)
    grid = (n_tok, n_hid)
    weights, indexes = pl.pallas_call(
        _router_kernel,
        grid=grid,
        in_specs=[
            pl.BlockSpec((TOK_BLK, D_MODEL), lambda i, j: (i, 0)),
            pl.BlockSpec((HID_BLK, D_MODEL), lambda i, j: (j, 0)),
            pl.BlockSpec((1, HID_BLK), lambda i, j: (0, j)),
            pl.BlockSpec((N_EXPERTS, HID_BLK), lambda i, j: (0, j)),
            pl.BlockSpec((1, N_EXPERTS), lambda i, j: (0, 0)),
        ],
        out_specs=[
            pl.BlockSpec((TOK_BLK, 2), lambda i, j: (i, 0)),
            pl.BlockSpec((TOK_BLK, 2), lambda i, j: (i, 0)),
        ],
        out_shape=[
            jax.ShapeDtypeStruct((TOKENS, 2), jnp.float32),
            jax.ShapeDtypeStruct((TOKENS, 2), jnp.int32),
        ],
        scratch_shapes=[pltpu.VMEM((TOK_BLK, N_EXPERTS), jnp.float32)],
        compiler_params=pltpu.CompilerParams(
            dimension_semantics=("parallel", "arbitrary"),
        ),
    )(x, W1, b1r, W2, b2r)
    return (weights, indexes)
